# Initial kernel scaffold; baseline (speedup 1.0000x reference)
import jax
import jax.numpy as jnp
from jax.experimental import pallas as pl


def kernel(user_embed_target_W, item_embed_target_W, user_embed_hist_W, item_embed_hist_W, user_embed_global, item_embed_global, proj_u_W, proj_u_b, proj_i_W, proj_i_b, user_idx, item_idx, user_hist, item_hist):
    raise NotImplementedError("write your pallas kernel here")


# speedup vs baseline: 2.0063x; 2.0063x over previous
import functools

import jax
import jax.numpy as jnp
from jax import lax
from jax.experimental import pallas as pl
from jax.experimental.pallas import tpu as pltpu
from jax.experimental.pallas import tpu_sc as plsc

_N_USERS = 100000
_N_ITEMS = 100000
_D = 64
_L = 50
_B = 4096

_NC = 2
_NS = 16
_NW = _NC * _NS

def _sc_mesh():
    return plsc.VectorSubcoreMesh(core_axis_name="c", subcore_axis_name="s",
                                  num_cores=_NC, num_subcores=_NS)


def _sc_hist_gather(hist_pack, user_idx, item_idx):
    n_per_w = _B // _NW

    @functools.partial(
        pl.kernel,
        mesh=_sc_mesh(),
        out_type=(
            jax.ShapeDtypeStruct((_B, 128), jnp.int32),
            jax.ShapeDtypeStruct((_B, 128), jnp.int32),
        ),
        scratch_types=[
            pltpu.VMEM((n_per_w,), jnp.int32),
            pltpu.VMEM((n_per_w, 128), jnp.int32),
            pltpu.SemaphoreType.DMA,
        ],
    )
    def k(h_hbm, ui_hbm, ii_hbm, ou_hbm, oi_hbm, idx_v, rows_v, sem):
        wid = lax.axis_index("s") * _NC + lax.axis_index("c")
        base = wid * n_per_w
        pltpu.sync_copy(ui_hbm.at[pl.ds(base, n_per_w)], idx_v)
        pltpu.async_copy(h_hbm.at[idx_v], rows_v, sem).wait()
        pltpu.sync_copy(rows_v, ou_hbm.at[pl.ds(base, n_per_w)])
        pltpu.sync_copy(ii_hbm.at[pl.ds(base, n_per_w)], idx_v)
        pltpu.async_copy(h_hbm.at[idx_v], rows_v, sem).wait()
        pltpu.sync_copy(rows_v, oi_hbm.at[pl.ds(base, n_per_w)])

    return k(hist_pack, user_idx, item_idx)


def _sc_embed_gather(u_pack, i_pack, flat_u, flat_i, user_idx, item_idx):
    n_big = _B * _L
    big_per_w = n_big // _NW
    chunk = 640
    n_chunks = big_per_w // chunk
    t_per_w = _B // _NW

    @functools.partial(
        pl.kernel,
        mesh=_sc_mesh(),
        out_type=(
            jax.ShapeDtypeStruct((n_big, 128), jnp.float32),
            jax.ShapeDtypeStruct((n_big, 128), jnp.float32),
            jax.ShapeDtypeStruct((_B, 128), jnp.float32),
            jax.ShapeDtypeStruct((_B, 128), jnp.float32),
        ),
        scratch_types=[
            pltpu.VMEM((t_per_w,), jnp.int32),
            pltpu.VMEM((t_per_w, 128), jnp.float32),
            pltpu.VMEM((chunk,), jnp.int32),
            pltpu.VMEM((chunk, 128), jnp.float32),
            pltpu.SemaphoreType.DMA,
        ],
    )
    def k(up_hbm, ip_hbm, fu_hbm, fi_hbm, ui_hbm, ii_hbm,
          gu_hbm, gi_hbm, ut_hbm, it_hbm, idx_t, rows_t, idx_b, rows_b, sem):
        wid = lax.axis_index("s") * _NC + lax.axis_index("c")

        tbase = wid * t_per_w
        pltpu.sync_copy(ui_hbm.at[pl.ds(tbase, t_per_w)], idx_t)
        pltpu.async_copy(up_hbm.at[idx_t], rows_t, sem).wait()
        pltpu.sync_copy(rows_t, ut_hbm.at[pl.ds(tbase, t_per_w)])
        pltpu.sync_copy(ii_hbm.at[pl.ds(tbase, t_per_w)], idx_t)
        pltpu.async_copy(ip_hbm.at[idx_t], rows_t, sem).wait()
        pltpu.sync_copy(rows_t, it_hbm.at[pl.ds(tbase, t_per_w)])

        @pl.loop(0, n_chunks)
        def _(j):
            base = wid * big_per_w + j * chunk
            pltpu.sync_copy(fu_hbm.at[pl.ds(base, chunk)], idx_b)
            pltpu.async_copy(ip_hbm.at[idx_b], rows_b, sem).wait()
            pltpu.sync_copy(rows_b, gu_hbm.at[pl.ds(base, chunk)])

        @pl.loop(0, n_chunks)
        def _(j):
            base = wid * big_per_w + j * chunk
            pltpu.sync_copy(fi_hbm.at[pl.ds(base, chunk)], idx_b)
            pltpu.async_copy(up_hbm.at[idx_b], rows_b, sem).wait()
            pltpu.sync_copy(rows_b, gi_hbm.at[pl.ds(base, chunk)])

    return k(u_pack, i_pack, flat_u, flat_i, user_idx, item_idx)


def _attn_body(gu_ref, gi_ref, idxu_ref, idxi_ref, uix_ref, iix_ref,
               wut_ref, bu_ref, gu_glob_ref, wit_ref, bi_ref, gi_glob_ref,
               hu_ref, hi_ref):
    def one_side(r_ref, idx_ref, tgt_ref, wt_ref, b_ref, g_ref, pad_id, o_ref):
        R = r_ref[...][:, :, _D:]
        bblk = R.shape[0]
        idx = idx_ref[...][:, :, None]
        tgt = tgt_ref[...][:, :, None]
        K = jnp.tanh(
            jnp.dot(R.reshape(bblk * _L, _D), wt_ref[...],
                    preferred_element_type=jnp.float32)
            + b_ref[...]
        ).reshape(bblk, _L, _D)
        g = g_ref[...].reshape(1, 1, _D)
        s = jnp.sum(K * g, axis=-1, keepdims=True) * (1.0 / 8.0)
        pad_mask = idx == pad_id
        mask = pad_mask | (idx == tgt)
        s = jnp.where(mask, -1e9, s)
        m = jnp.max(s, axis=1, keepdims=True)
        e = jnp.exp(s - m)
        w = e / jnp.sum(e, axis=1, keepdims=True)
        w = jnp.where(pad_mask, 0.0, w)
        o_ref[...] = jnp.sum(R * w, axis=1)

    one_side(gu_ref, idxu_ref, iix_ref, wut_ref, bu_ref, gu_glob_ref,
             _N_ITEMS, hu_ref)
    one_side(gi_ref, idxi_ref, uix_ref, wit_ref, bi_ref, gi_glob_ref,
             _N_USERS, hi_ref)


def _tc_attention(g_u, g_i, idx_u, idx_i, user_idx2, item_idx2,
                  wut, bu, gu_glob, wit, bi, gi_glob):
    bblk = 128
    grid = (_B // bblk,)
    full = lambda i: (0, 0)
    return pl.pallas_call(
        _attn_body,
        grid=grid,
        in_specs=[
            pl.BlockSpec((bblk, _L, 128), lambda i: (i, 0, 0)),
            pl.BlockSpec((bblk, _L, 128), lambda i: (i, 0, 0)),
            pl.BlockSpec((bblk, _L), lambda i: (i, 0)),
            pl.BlockSpec((bblk, _L), lambda i: (i, 0)),
            pl.BlockSpec((bblk, 1), lambda i: (i, 0)),
            pl.BlockSpec((bblk, 1), lambda i: (i, 0)),
            pl.BlockSpec((_D, _D), full),
            pl.BlockSpec((1, _D), full),
            pl.BlockSpec((1, _D), full),
            pl.BlockSpec((_D, _D), full),
            pl.BlockSpec((1, _D), full),
            pl.BlockSpec((1, _D), full),
        ],
        out_specs=[
            pl.BlockSpec((bblk, _D), lambda i: (i, 0)),
            pl.BlockSpec((bblk, _D), lambda i: (i, 0)),
        ],
        out_shape=[
            jax.ShapeDtypeStruct((_B, _D), jnp.float32),
            jax.ShapeDtypeStruct((_B, _D), jnp.float32),
        ],
        compiler_params=pltpu.CompilerParams(
            dimension_semantics=("parallel",),
        ),
    )(g_u, g_i, idx_u, idx_i, user_idx2, item_idx2,
      wut, bu, gu_glob, wit, bi, gi_glob)


def kernel(user_embed_target_W, item_embed_target_W, user_embed_hist_W,
           item_embed_hist_W, user_embed_global, item_embed_global,
           proj_u_W, proj_u_b, proj_i_W, proj_i_b,
           user_idx, item_idx, user_hist, item_hist):
    zpad = jnp.zeros((100000, 64 - _L), jnp.int32)
    hist_pack = jnp.concatenate([user_hist, zpad, item_hist, zpad], axis=1)
    u_pack = jnp.concatenate([user_embed_target_W, user_embed_hist_W], axis=1)
    i_pack = jnp.concatenate([item_embed_target_W, item_embed_hist_W], axis=1)

    ref_u_pad, ref_i_pad = _sc_hist_gather(hist_pack, user_idx, item_idx)

    idx_u = ref_u_pad[:, :_L]
    idx_i = ref_i_pad[:, 64:64 + _L]
    flat_u = idx_u.reshape(-1)
    flat_i = idx_i.reshape(-1)

    g_u, g_i, u_t128, i_t128 = _sc_embed_gather(
        u_pack, i_pack, flat_u, flat_i, user_idx, item_idx)
    u_t = u_t128[:, :_D]
    i_t = i_t128[:, :_D]

    hu, hi = _tc_attention(
        g_u.reshape(_B, _L, 128), g_i.reshape(_B, _L, 128),
        idx_u, idx_i,
        user_idx.reshape(_B, 1), item_idx.reshape(_B, 1),
        proj_u_W.T, proj_u_b.reshape(1, _D), user_embed_global.reshape(1, _D),
        proj_i_W.T, proj_i_b.reshape(1, _D), item_embed_global.reshape(1, _D))

    id_cat = jnp.concatenate([u_t, i_t], axis=-1)
    hist_cat = jnp.concatenate([hu, hi], axis=-1)
    user_cat = jnp.concatenate([u_t, hu], axis=-1)
    item_cat = jnp.concatenate([i_t, hi], axis=-1)
    return (id_cat, hist_cat, user_cat, item_cat)
